# baseline (speedup 1.0000x reference)
import functools

import jax
import jax.numpy as jnp
from jax.experimental import pallas as pl
import jax.experimental.pallas.tpu as pltpu

_LAMBD = 0.5
_SIGMA = 0.35
_TEMP = 0.07
_TOP_P = 0.9
_TOP_K = 50

_NEG = -3e38


def _sims_kernel(embeds_ref, qsizes_ref, cat_ref, csizes_ref, cstsizes_ref,
                 out_ref):
    q = embeds_ref[...]
    q = q / jnp.maximum(jnp.sqrt(jnp.sum(q * q, axis=1, keepdims=True)),
                        1e-12)
    x = cat_ref[...]
    x = x * (1.0 / jnp.maximum(jnp.sqrt(jnp.sum(x * x, axis=1, keepdims=True)),
                               1e-12))
    sem = jax.lax.dot_general(
        q, x, (((1,), (1,)), ((), ())),
        preferred_element_type=jnp.float32,
    )

    qs = qsizes_ref[...]
    cs = csizes_ref[...]
    cst = cstsizes_ref[...]
    b2 = jnp.sum(qs * qs, axis=1, keepdims=True)
    a2 = jnp.sum(cst * cst, axis=0, keepdims=True)
    dot = jax.lax.dot_general(
        qs, cs, (((1,), (1,)), ((), ())),
        preferred_element_type=jnp.float32,
    )
    size_diffs = a2 + b2 - 2.0 * dot
    size_sim = jnp.exp(-size_diffs / (2.0 * _SIGMA * _SIGMA))

    out_ref[...] = (_LAMBD * sem + (1.0 - _LAMBD) * size_sim) / _TEMP


def _topp_kernel(sims_ref, out_ref, cur_ref, pool_ref, flat_ref, *, k_valid):
    r, padk = sims_ref.shape
    nseg = padk // 128
    col = jax.lax.broadcasted_iota(jnp.int32, (r, padk), 1)
    v = jnp.where(col < k_valid, sims_ref[...], _NEG)
    pool_ref[...] = jnp.full((r, 64, 128), _NEG, jnp.float32)

    lane64 = jax.lax.broadcasted_iota(jnp.int32, (r, 64), 1)

    def pop_from(c, j):
        mj = jnp.max(c, axis=1)
        pool_ref[:, pl.ds(j, 1), :] = mj[:, None, :]
        cur_ref[...] = jnp.where(c == mj[:, None, :], _NEG, c)

    def pop_level(j):
        pop_from(cur_ref[...], j)

    def top50_fast():
        def body(i, carry):
            vals, flat = carry
            m = jnp.max(flat, axis=1, keepdims=True)
            vals = jnp.where(lane64 == i, m, vals)
            flat = jnp.where(flat == m, _NEG, flat)
            return vals, flat

        vals, _ = jax.lax.fori_loop(
            0, _TOP_K, body,
            (jnp.full((r, 64), _NEG, jnp.float32),
             pool_ref[:, :4, :].reshape(r, 4 * 128)))
        return vals

    def top50_of_pool():
        flat_ref[...] = pool_ref[...].reshape(r, 64 * 128)

        def body(i, vals):
            flat = flat_ref[...]
            m = jnp.max(flat, axis=1, keepdims=True)
            vals = jnp.where(lane64 == i, m, vals)
            flat_ref[...] = jnp.where(flat == m, _NEG, flat)
            return vals

        return jax.lax.fori_loop(
            0, _TOP_K, body, jnp.full((r, 64), _NEG, jnp.float32))

    pop_from(v.reshape(r, nseg, 128), 0)
    for j in range(1, 4):
        pop_level(j)
    vals = top50_fast()

    def unfinished(vals):
        gmax = jnp.max(jnp.max(cur_ref[...], axis=1), axis=1,
                       keepdims=True)
        t50 = vals[:, _TOP_K - 1:_TOP_K]
        return jnp.any(gmax >= t50)

    def wbody(state):
        j, _, _ = state
        pop_level(j)
        vals = top50_of_pool()
        return j + 1, vals, unfinished(vals)

    def wcond(state):
        return state[2]

    _, vals, _ = jax.lax.while_loop(
        wcond, wbody, (4, vals, unfinished(vals)))

    in_topk = lane64 < _TOP_K
    vmax = vals[:, 0:1]
    e = jnp.where(in_topk, jnp.exp(vals - vmax), 0.0)
    z = jnp.sum(e, axis=1, keepdims=True)
    p = e / z
    ti = jax.lax.broadcasted_iota(jnp.int32, (64, 64), 0)
    tj = jax.lax.broadcasted_iota(jnp.int32, (64, 64), 1)
    tri = (ti <= tj).astype(jnp.float32)
    cum = jax.lax.dot_general(
        p, tri, (((1,), (0,)), ((), ())),
        preferred_element_type=jnp.float32,
        precision=jax.lax.Precision.HIGHEST,
    )
    keep = jnp.logical_and(cum - p <= _TOP_P, in_topk)
    w = jnp.sum(jnp.where(keep, p, 0.0), axis=1, keepdims=True)
    tkeep = jnp.min(jnp.where(keep, vals, 3e38),
                    axis=1, keepdims=True)

    scale = 1.0 / (z * w)
    out = jnp.where(v >= tkeep, jnp.exp(v - vmax) * scale, 0.0)
    out_ref[...] = out[:, : out_ref.shape[1]]


def kernel(embeds, query_sizes, catalog_embeds, catalog_sizes):
    qn, d = embeds.shape
    kn = catalog_embeds.shape[0]

    blk = 4096
    nb = -(-kn // blk)
    padk = nb * blk

    sims = pl.pallas_call(
        _sims_kernel,
        grid=(nb,),
        in_specs=[
            pl.BlockSpec((qn, d), lambda i: (0, 0)),
            pl.BlockSpec((qn, 3), lambda i: (0, 0)),
            pl.BlockSpec((blk, d), lambda i: (i, 0)),
            pl.BlockSpec((blk, 3), lambda i: (i, 0)),
            pl.BlockSpec((3, blk), lambda i: (0, i)),
        ],
        out_specs=pl.BlockSpec((qn, blk), lambda i: (0, i)),
        out_shape=jax.ShapeDtypeStruct((qn, padk), jnp.float32),
    )(embeds, query_sizes, catalog_embeds, catalog_sizes, catalog_sizes.T)

    rblk = 8
    probs = pl.pallas_call(
        functools.partial(_topp_kernel, k_valid=kn),
        grid=(qn // rblk,),
        in_specs=[pl.BlockSpec((rblk, padk), lambda i: (i, 0))],
        out_specs=pl.BlockSpec((rblk, kn), lambda i: (i, 0)),
        out_shape=jax.ShapeDtypeStruct((qn, kn), jnp.float32),
        scratch_shapes=[
            pltpu.VMEM((rblk, padk // 128, 128), jnp.float32),
            pltpu.VMEM((rblk, 64, 128), jnp.float32),
            pltpu.VMEM((rblk, 64 * 128), jnp.float32),
        ],
    )(sims)
    return probs


# speedup vs baseline: 1.0893x; 1.0893x over previous
import functools

import jax
import jax.numpy as jnp
from jax.experimental import pallas as pl
import jax.experimental.pallas.tpu as pltpu

_LAMBD = 0.5
_SIGMA = 0.35
_TEMP = 0.07
_TOP_P = 0.9
_TOP_K = 50

_NEG = -3e38


def _sims_kernel(embeds_ref, qsizes_ref, cat_ref, csizes_ref, cstsizes_ref,
                 out_ref):
    q = embeds_ref[...]
    q = q / jnp.maximum(jnp.sqrt(jnp.sum(q * q, axis=1, keepdims=True)),
                        1e-12)
    x = cat_ref[...]
    x = x * (1.0 / jnp.maximum(jnp.sqrt(jnp.sum(x * x, axis=1, keepdims=True)),
                               1e-12))
    sem = jax.lax.dot_general(
        q, x, (((1,), (1,)), ((), ())),
        preferred_element_type=jnp.float32,
    )

    qs = qsizes_ref[...]
    cs = csizes_ref[...]
    cst = cstsizes_ref[...]
    b2 = jnp.sum(qs * qs, axis=1, keepdims=True)
    a2 = jnp.sum(cst * cst, axis=0, keepdims=True)
    dot = jax.lax.dot_general(
        qs, cs, (((1,), (1,)), ((), ())),
        preferred_element_type=jnp.float32,
    )
    size_diffs = a2 + b2 - 2.0 * dot
    size_sim = jnp.exp(-size_diffs / (2.0 * _SIGMA * _SIGMA))

    out_ref[...] = (_LAMBD * sem + (1.0 - _LAMBD) * size_sim) / _TEMP


def _topp_kernel(sims_ref, out_ref, cur_ref, pool_ref, flat_ref, *, k_valid):
    r, padk = sims_ref.shape
    nseg = padk // 128
    col = jax.lax.broadcasted_iota(jnp.int32, (r, padk), 1)
    v = jnp.where(col < k_valid, sims_ref[...], _NEG)
    pool_ref[...] = jnp.full((r, 64, 128), _NEG, jnp.float32)

    lane64 = jax.lax.broadcasted_iota(jnp.int32, (r, 64), 1)

    def pop_from(c, j):
        mj = jnp.max(c, axis=1)
        pool_ref[:, pl.ds(j, 1), :] = mj[:, None, :]
        cur_ref[...] = jnp.where(c == mj[:, None, :], _NEG, c)

    def pop_level(j):
        pop_from(cur_ref[...], j)

    def top50_fast():
        def body(i, carry):
            vals, flat = carry
            m = jnp.max(flat, axis=1, keepdims=True)
            vals = jnp.where(lane64 == i, m, vals)
            flat = jnp.where(flat == m, _NEG, flat)
            return vals, flat

        vals, _ = jax.lax.fori_loop(
            0, _TOP_K, body,
            (jnp.full((r, 64), _NEG, jnp.float32),
             pool_ref[:, :4, :].reshape(r, 4 * 128)))
        return vals

    def top50_of_pool():
        flat_ref[...] = pool_ref[...].reshape(r, 64 * 128)

        def body(i, vals):
            flat = flat_ref[...]
            m = jnp.max(flat, axis=1, keepdims=True)
            vals = jnp.where(lane64 == i, m, vals)
            flat_ref[...] = jnp.where(flat == m, _NEG, flat)
            return vals

        return jax.lax.fori_loop(
            0, _TOP_K, body, jnp.full((r, 64), _NEG, jnp.float32))

    pop_from(v.reshape(r, nseg, 128), 0)
    for j in range(1, 4):
        pop_level(j)
    vals = top50_fast()

    def unfinished(vals):
        gmax = jnp.max(jnp.max(cur_ref[...], axis=1), axis=1,
                       keepdims=True)
        t50 = vals[:, _TOP_K - 1:_TOP_K]
        return jnp.any(gmax >= t50)

    def wbody(state):
        j, _, _ = state
        pop_level(j)
        vals = top50_of_pool()
        return j + 1, vals, unfinished(vals)

    def wcond(state):
        return state[2]

    _, vals, _ = jax.lax.while_loop(
        wcond, wbody, (4, vals, unfinished(vals)))

    in_topk = lane64 < _TOP_K
    vmax = vals[:, 0:1]
    e = jnp.where(in_topk, jnp.exp(vals - vmax), 0.0)
    z = jnp.sum(e, axis=1, keepdims=True)
    p = e / z
    ti = jax.lax.broadcasted_iota(jnp.int32, (64, 64), 0)
    tj = jax.lax.broadcasted_iota(jnp.int32, (64, 64), 1)
    tri = (ti <= tj).astype(jnp.float32)
    cum = jax.lax.dot_general(
        p, tri, (((1,), (0,)), ((), ())),
        preferred_element_type=jnp.float32,
        precision=jax.lax.Precision.HIGHEST,
    )
    keep = jnp.logical_and(cum - p <= _TOP_P, in_topk)
    w = jnp.sum(jnp.where(keep, p, 0.0), axis=1, keepdims=True)
    tkeep = jnp.min(jnp.where(keep, vals, 3e38),
                    axis=1, keepdims=True)

    scale = 1.0 / (z * w)
    out = jnp.where(v >= tkeep, jnp.exp(v - vmax) * scale, 0.0)
    out_ref[...] = out[:, : out_ref.shape[1]]


def kernel(embeds, query_sizes, catalog_embeds, catalog_sizes):
    qn, d = embeds.shape
    kn = catalog_embeds.shape[0]

    blk = 4096
    nb = -(-kn // blk)
    padk = nb * blk

    sims = pl.pallas_call(
        _sims_kernel,
        grid=(nb,),
        in_specs=[
            pl.BlockSpec((qn, d), lambda i: (0, 0)),
            pl.BlockSpec((qn, 3), lambda i: (0, 0)),
            pl.BlockSpec((blk, d), lambda i: (i, 0)),
            pl.BlockSpec((blk, 3), lambda i: (i, 0)),
            pl.BlockSpec((3, blk), lambda i: (0, i)),
        ],
        out_specs=pl.BlockSpec((qn, blk), lambda i: (0, i)),
        out_shape=jax.ShapeDtypeStruct((qn, padk), jnp.float32),
    )(embeds, query_sizes, catalog_embeds, catalog_sizes, catalog_sizes.T)

    rblk = 16
    probs = pl.pallas_call(
        functools.partial(_topp_kernel, k_valid=kn),
        grid=(qn // rblk,),
        in_specs=[pl.BlockSpec((rblk, padk), lambda i: (i, 0))],
        out_specs=pl.BlockSpec((rblk, kn), lambda i: (i, 0)),
        out_shape=jax.ShapeDtypeStruct((qn, kn), jnp.float32),
        scratch_shapes=[
            pltpu.VMEM((rblk, padk // 128, 128), jnp.float32),
            pltpu.VMEM((rblk, 64, 128), jnp.float32),
            pltpu.VMEM((rblk, 64 * 128), jnp.float32),
        ],
    )(sims)
    return probs
